# baseline (speedup 1.0000x reference)
import jax
import jax.numpy as jnp
from jax import lax
from jax.experimental import pallas as pl
from jax.experimental.pallas import tpu as pltpu
from jax.experimental.pallas import tpu_sc as plsc

ROWS, COLS = 16384, 128
NUM_CORES, NUM_SUBCORES = 2, 16
NUM_WORKERS = NUM_CORES * NUM_SUBCORES
ROWS_PER_WORKER = ROWS // NUM_WORKERS
BLK = 64
CHUNK = 512
NCHUNK = ROWS // CHUNK
NBUF = 32


def _tc_reduce_body(x_hbm, blk_ref, bufs, *sems):
    for k in range(NBUF):
        pltpu.make_async_copy(
            x_hbm.at[pl.ds(k * CHUNK, CHUNK)], bufs.at[k], sems[k]
        ).start()

    acc = jnp.zeros((1, COLS), jnp.float32)
    for k in range(NCHUNK):
        b = k % NBUF
        pltpu.make_async_copy(
            x_hbm.at[pl.ds(k * CHUNK, CHUNK)], bufs.at[b], sems[b]
        ).wait()
        a = bufs[b].reshape(CHUNK // 8, 8, COLS)
        if k + NBUF < NCHUNK:
            pltpu.make_async_copy(
                x_hbm.at[pl.ds((k + NBUF) * CHUNK, CHUNK)], bufs.at[b], sems[b]
            ).start()
        while a.shape[0] > 1:
            h = a.shape[0] // 2
            a = a[:h] + a[h:]
        acc = acc + jnp.sum(a[0], axis=0, keepdims=True)

    col = lax.broadcasted_iota(jnp.int32, (1, COLS), 1)
    m = jnp.min(jnp.where(acc >= 0.0, col, COLS))
    idx = jnp.where(m >= COLS, 0, m)
    blk_ref[...] = jnp.where(
        lax.broadcasted_iota(jnp.int32, (BLK, COLS), 1) == idx, 1.0, -1.0
    ).astype(jnp.float32)


_tc_reduce = pl.pallas_call(
    _tc_reduce_body,
    in_specs=[pl.BlockSpec(memory_space=pl.MemorySpace.ANY)],
    out_shape=jax.ShapeDtypeStruct((BLK, COLS), jnp.float32),
    scratch_shapes=[pltpu.VMEM((NBUF, CHUNK, COLS), jnp.float32)]
    + [pltpu.SemaphoreType.DMA] * NBUF,
)


_MESH = plsc.VectorSubcoreMesh(
    core_axis_name="c", subcore_axis_name="s",
    num_cores=NUM_CORES, num_subcores=NUM_SUBCORES,
)


def _sc_write_body(blk_hbm, out_hbm, blk_v, sem):
    cid = lax.axis_index("c")
    sid = lax.axis_index("s")
    wid = cid * NUM_SUBCORES + sid
    base = wid * ROWS_PER_WORKER

    pltpu.sync_copy(blk_hbm, blk_v)
    copies = [
        pltpu.make_async_copy(
            blk_v, out_hbm.at[pl.ds(base + b * BLK, BLK)], sem
        )
        for b in range(ROWS_PER_WORKER // BLK)
    ]
    for c in copies:
        c.start()
    for c in copies:
        c.wait()


_sc_write = pl.kernel(
    _sc_write_body,
    out_type=jax.ShapeDtypeStruct((ROWS, COLS), jnp.float32),
    mesh=_MESH,
    compiler_params=pltpu.CompilerParams(needs_layout_passes=False),
    scratch_types=[
        pltpu.VMEM((BLK, COLS), jnp.float32),
        pltpu.SemaphoreType.DMA,
    ],
)


@jax.jit
def kernel(x):
    return _sc_write(_tc_reduce(x))


# speedup vs baseline: 1.0101x; 1.0101x over previous
import jax
import jax.numpy as jnp
from jax import lax
from jax.experimental import pallas as pl
from jax.experimental.pallas import tpu as pltpu
from jax.experimental.pallas import tpu_sc as plsc

ROWS, COLS = 16384, 128
NUM_CORES, NUM_SUBCORES = 2, 16
NUM_WORKERS = NUM_CORES * NUM_SUBCORES
ROWS_PER_WORKER = ROWS // NUM_WORKERS
BLK = 64
CHUNK = 512
NCHUNK = ROWS // CHUNK
NBUF = 16


def _tc_reduce_body(x_hbm, blk_ref, bufs, *sems):
    for k in range(NBUF):
        pltpu.make_async_copy(
            x_hbm.at[pl.ds(k * CHUNK, CHUNK)], bufs.at[k], sems[k]
        ).start()

    acc = jnp.zeros((1, COLS), jnp.float32)
    for k in range(NCHUNK):
        b = k % NBUF
        pltpu.make_async_copy(
            x_hbm.at[pl.ds(k * CHUNK, CHUNK)], bufs.at[b], sems[b]
        ).wait()
        a = bufs[b].reshape(CHUNK // 8, 8, COLS)
        if k + NBUF < NCHUNK:
            pltpu.make_async_copy(
                x_hbm.at[pl.ds((k + NBUF) * CHUNK, CHUNK)], bufs.at[b], sems[b]
            ).start()
        while a.shape[0] > 1:
            h = a.shape[0] // 2
            a = a[:h] + a[h:]
        acc = acc + jnp.sum(a[0], axis=0, keepdims=True)

    col = lax.broadcasted_iota(jnp.int32, (1, COLS), 1)
    m = jnp.min(jnp.where(acc >= 0.0, col, COLS))
    idx = jnp.where(m >= COLS, 0, m)
    blk_ref[...] = jnp.where(
        lax.broadcasted_iota(jnp.int32, (BLK, COLS), 1) == idx, 1.0, -1.0
    ).astype(jnp.float32)


_tc_reduce = pl.pallas_call(
    _tc_reduce_body,
    in_specs=[pl.BlockSpec(memory_space=pl.MemorySpace.ANY)],
    out_shape=jax.ShapeDtypeStruct((BLK, COLS), jnp.float32),
    scratch_shapes=[pltpu.VMEM((NBUF, CHUNK, COLS), jnp.float32)]
    + [pltpu.SemaphoreType.DMA] * NBUF,
)


_MESH = plsc.VectorSubcoreMesh(
    core_axis_name="c", subcore_axis_name="s",
    num_cores=NUM_CORES, num_subcores=NUM_SUBCORES,
)


def _sc_write_body(blk_hbm, out_hbm, blk_v, sem):
    cid = lax.axis_index("c")
    sid = lax.axis_index("s")
    wid = cid * NUM_SUBCORES + sid
    base = wid * ROWS_PER_WORKER

    pltpu.sync_copy(blk_hbm, blk_v)
    copies = [
        pltpu.make_async_copy(
            blk_v, out_hbm.at[pl.ds(base + b * BLK, BLK)], sem
        )
        for b in range(ROWS_PER_WORKER // BLK)
    ]
    for c in copies:
        c.start()
    for c in copies:
        c.wait()


_sc_write = pl.kernel(
    _sc_write_body,
    out_type=jax.ShapeDtypeStruct((ROWS, COLS), jnp.float32),
    mesh=_MESH,
    compiler_params=pltpu.CompilerParams(needs_layout_passes=False),
    scratch_types=[
        pltpu.VMEM((BLK, COLS), jnp.float32),
        pltpu.SemaphoreType.DMA,
    ],
)


@jax.jit
def kernel(x):
    return _sc_write(_tc_reduce(x))
